# baseline (speedup 1.0000x reference)
import functools

import numpy as np
import jax
import jax.numpy as jnp
from jax import lax
from jax.experimental import pallas as pl
from jax.experimental.pallas import tpu as pltpu
from jax.experimental.pallas import tpu_sc as plsc

_N = 10000
_E = 160000
_D = 256
_HALF = 128
_NCODE = 32
_CHUNK = 128
_NCHUNK = _E // _CHUNK
_NSUB = 16
_NCORE = 2
_RS0 = 632
_RSLAST = _N - (_NSUB - 1) * _RS0
_R = 1000
_GRID = _N // _R
_BOND_ROWS = [7, 7, 3, 3, 3]
_ET = 24

_EPS = 1e-5


def _make_selector() -> np.ndarray:
    off = np.cumsum([0] + _BOND_ROWS[:-1])
    s = np.zeros((_NCODE, _ET), np.float32)
    for c in range(_NCODE):
        for i in range(5):
            s[c, off[i] + ((c >> i) & 1)] += 1.0
    return s


_SEL = _make_selector()

_f32 = jnp.float32


def _zero_accum(sid, zer_hbm, accum, r0=_RS0, rlast=_RSLAST):
    start = pl.multiple_of(sid * r0, 8)

    @pl.when(sid < _NSUB - 1)
    def _():
        pltpu.sync_copy(zer_hbm.at[pl.ds(0, r0)], accum.at[pl.ds(start, r0)])

    @pl.when(sid == _NSUB - 1)
    def _():
        pltpu.sync_copy(zer_hbm.at[pl.ds(0, rlast)],
                        accum.at[pl.ds(start, rlast)])


def _copy_out(sid, base, accum, out_hbm, r0=_RS0, rlast=_RSLAST):
    start = pl.multiple_of(sid * r0, 8)
    dst0 = pl.multiple_of(base + sid * r0, 8)

    @pl.when(sid < _NSUB - 1)
    def _():
        pltpu.sync_copy(accum.at[pl.ds(start, r0)],
                        out_hbm.at[pl.ds(dst0, r0)])

    @pl.when(sid == _NSUB - 1)
    def _():
        pltpu.sync_copy(accum.at[pl.ds(start, rlast)],
                        out_hbm.at[pl.ds(dst0, rlast)])


_NB_S = 4
_CHUNK_S = 64
_NB_C = 3


def _ring_loop(nb, stride, wid, a_hbm, b_hbm, table_hbm, acc,
               abuf, bbuf, sbuf, rbuf, asem, bsem, gsem, ssem,
               transform_a, transform_b, chunk=_CHUNK):
    nchunk = _E // chunk
    nouter = nchunk // (nb * stride)

    def outer(k, carry):
        def cbase(b):
            return ((k * nb + b) * stride + wid) * chunk

        for b in range(nb):
            pltpu.async_copy(a_hbm.at[pl.ds(cbase(b), chunk)],
                             abuf[b], asem[b])
            pltpu.async_copy(b_hbm.at[pl.ds(cbase(b), chunk)],
                             bbuf[b], bsem[b])
        for b in range(nb):
            pltpu.make_async_copy(a_hbm.at[pl.ds(cbase(b), chunk)],
                                  abuf[b], asem[b]).wait()
            pltpu.make_async_copy(b_hbm.at[pl.ds(cbase(b), chunk)],
                                  bbuf[b], bsem[b]).wait()
            transform_a(abuf[b], bbuf[b])
            @pl.when(k > 0)
            def _(b=b):
                pltpu.make_async_copy(rbuf[b], acc.at[sbuf[b]],
                                      ssem[b]).wait()
            pltpu.async_copy(table_hbm.at[abuf[b]], rbuf[b], gsem[b])
        for b in range(nb):
            pltpu.make_async_copy(table_hbm.at[abuf[b]], rbuf[b],
                                  gsem[b]).wait()
            transform_b(bbuf[b])
            for j in range(chunk // 16):
                sl = pl.ds(j * 16, 16)
                sbuf[b][sl] = bbuf[b][sl]
            pltpu.async_copy(rbuf[b], acc.at[sbuf[b]], ssem[b], add=True)
        return carry

    lax.fori_loop(0, nouter, outer, 0)
    for b in range(nb):
        pltpu.make_async_copy(rbuf[b], acc.at[sbuf[b]], ssem[b]).wait()

    tail = nchunk - nouter * nb * stride

    @pl.when(wid < tail)
    def _():
        base = (nouter * nb * stride + wid) * chunk
        pltpu.sync_copy(a_hbm.at[pl.ds(base, chunk)], abuf[0])
        pltpu.sync_copy(b_hbm.at[pl.ds(base, chunk)], bbuf[0])
        transform_a(abuf[0], bbuf[0])
        pltpu.async_copy(table_hbm.at[abuf[0]], rbuf[0], gsem[0]).wait()
        transform_b(bbuf[0])
        pltpu.sync_copy(rbuf[0], acc.at[bbuf[0]], add=True)


def _sc_spmv_body(src_hbm, dst_hbm, h2n_hbm, zer_hbm, out_hbm, *scr):
    nb = _NB_S
    abuf, bbuf, sbuf = scr[:nb], scr[nb:2 * nb], scr[2 * nb:3 * nb]
    rbuf = scr[3 * nb:4 * nb]
    accum = scr[4 * nb]
    sems = scr[4 * nb + 1:]
    asem, bsem = sems[:nb], sems[nb:2 * nb]
    gsem, ssem = sems[2 * nb:3 * nb], sems[3 * nb:4 * nb]

    cid = lax.axis_index("c")
    sid = lax.axis_index("s")
    _zero_accum(sid, zer_hbm, accum)
    plsc.subcore_barrier()

    row_off = cid * _N

    def add_off(a, b_unused):
        for j in range(_CHUNK_S // 16):
            sl = pl.ds(j * 16, 16)
            a[sl] = a[sl] + row_off

    _ring_loop(nb, _NSUB, sid, src_hbm, dst_hbm, h2n_hbm, accum,
               abuf, bbuf, sbuf, rbuf, asem, bsem, gsem, ssem,
               add_off, lambda b: None, chunk=_CHUNK_S)

    plsc.subcore_barrier()
    _copy_out(sid, cid * _N, accum, out_hbm)


_sc_cache = {}


def _get_sc_kernels():
    if 'spmv' not in _sc_cache:
        mesh = plsc.VectorSubcoreMesh(
            core_axis_name="c", subcore_axis_name="s",
            num_cores=_NCORE, num_subcores=_NSUB)
        _sc_cache['spmv'] = functools.partial(
            pl.kernel,
            out_type=jax.ShapeDtypeStruct((_NCORE * _N, _HALF), _f32),
            mesh=mesh,
            scratch_types=(
                [pltpu.VMEM((_CHUNK_S,), jnp.int32)] * (3 * _NB_S)
                + [pltpu.VMEM((_CHUNK_S, _HALF), _f32)] * _NB_S
                + [pltpu.VMEM_SHARED((_N, _HALF), _f32)]
                + [pltpu.SemaphoreType.DMA] * (4 * _NB_S)
            ),
        )(_sc_spmv_body)
        _sc_cache['counts'] = functools.partial(
            pl.kernel,
            out_type=jax.ShapeDtypeStruct((_NCORE * _N, _HALF), _f32),
            mesh=mesh,
            scratch_types=(
                [pltpu.VMEM((_CHUNK,), jnp.int32)] * (3 * _NB_C)
                + [pltpu.VMEM((_CHUNK, _HALF), _f32)] * _NB_C
                + [pltpu.VMEM_SHARED((_N, _HALF), _f32)]
                + [pltpu.SemaphoreType.DMA] * (4 * _NB_C)
            ),
        )(_sc_counts_body)
    return _sc_cache['spmv'], _sc_cache['counts']


_QROWS = 2504
_QR0 = 160
_QRLAST = _QROWS - (_NSUB - 1) * _QR0


def _sc_counts_body(code_hbm, dst_hbm, zer_hbm, id128_hbm, out_hbm, *scr):
    nb = _NB_C
    abuf, bbuf, sbuf = scr[:nb], scr[nb:2 * nb], scr[2 * nb:3 * nb]
    rbuf = scr[3 * nb:4 * nb]
    accum = scr[4 * nb]
    sems = scr[4 * nb + 1:]
    asem, bsem = sems[:nb], sems[nb:2 * nb]
    gsem, ssem = sems[2 * nb:3 * nb], sems[3 * nb:4 * nb]

    cid = lax.axis_index("c")
    sid = lax.axis_index("s")
    _zero_accum(sid, zer_hbm, accum)
    plsc.subcore_barrier()

    wid = sid * _NCORE + cid
    tab_off = wid * _NCODE

    def tf_a(a, b):
        for j in range(_CHUNK // 16):
            sl = pl.ds(j * 16, 16)
            a[sl] = a[sl] + tab_off

    _ring_loop(nb, _NSUB * _NCORE, wid, code_hbm, dst_hbm, id128_hbm, accum,
               abuf, bbuf, sbuf, rbuf, asem, bsem, gsem, ssem,
               tf_a, lambda b: None)

    plsc.subcore_barrier()
    _copy_out(sid, cid * _N, accum, out_hbm)


def _bn_relu(y, stats, gamma, beta):
    mu = stats[0:1, :] * (1.0 / _N)
    var = stats[1:2, :] * (1.0 / _N) - mu * mu
    return jnp.maximum(gamma * (y - mu) * lax.rsqrt(var + _EPS) + beta, 0.0)


def _layer_common(i, sp_lo, sp_hi, h_lo, h_hi, c_lo, c_hi, sel, es, wt, b,
                  y_scr, st_scr):
    t = jnp.dot(sel[...], es[...], preferred_element_type=_f32,
                precision=lax.Precision.HIGHEST)
    cb = c_lo[...][:, :_NCODE] + c_hi[...][:, :_NCODE]
    emb = jnp.dot(cb, t, preferred_element_type=_f32,
                  precision=lax.Precision.HIGHEST) + t[0:1, :]
    aggr = jnp.concatenate(
        [sp_lo[...] + h_lo[...], sp_hi[...] + h_hi[...]], axis=1) + emb
    y = jnp.dot(aggr.astype(jnp.bfloat16), wt[...].astype(jnp.bfloat16),
                preferred_element_type=_f32) + b[...]
    y_scr[pl.ds(pl.multiple_of(i * _R, 8), _R), :] = y
    st = jnp.concatenate(
        [jnp.sum(y, axis=0, keepdims=True),
         jnp.sum(y * y, axis=0, keepdims=True)], axis=0)

    @pl.when(i == 0)
    def _():
        st_scr[...] = st

    @pl.when(i > 0)
    def _():
        st_scr[...] += st


def _tc_layer_body(sp_lo, sp_hi, h_lo, h_hi, c_lo, c_hi, sel, es, wt, b,
                   gamma, beta, out, y_scr, st_scr):
    i = pl.program_id(0)
    _layer_common(i, sp_lo, sp_hi, h_lo, h_hi, c_lo, c_hi, sel, es, wt, b,
                  y_scr, st_scr)

    @pl.when(i == _GRID - 1)
    def _():
        def norm_blk(j, carry):
            r0 = pl.multiple_of(j * _R, 8)
            h = _bn_relu(y_scr[pl.ds(r0, _R), :], st_scr[...],
                         gamma[...], beta[...])
            out[pl.ds(r0, _R), :] = h[:, :_HALF]
            out[pl.ds(_N + r0, _R), :] = h[:, _HALF:]
            return carry

        lax.fori_loop(0, _GRID, norm_blk, 0)


def _tc_final_body(sp_lo, sp_hi, h_lo, h_hi, c_lo, c_hi, sel, es, wt, b,
                   gamma, beta, wt_o, b_o, out, y_scr, st_scr):
    i = pl.program_id(0)
    _layer_common(i, sp_lo, sp_hi, h_lo, h_hi, c_lo, c_hi, sel, es, wt, b,
                  y_scr, st_scr)

    @pl.when(i == _GRID - 1)
    def _():
        def norm_blk(j, carry):
            r0 = pl.multiple_of(j * _R, 8)
            h = _bn_relu(y_scr[pl.ds(r0, _R), :], st_scr[...],
                         gamma[...], beta[...])
            out[pl.ds(r0, _R), :] = jnp.dot(
                h.astype(jnp.bfloat16), wt_o[...].astype(jnp.bfloat16),
                preferred_element_type=_f32) + b_o[...]
            return carry

        lax.fori_loop(0, _GRID, norm_blk, 0)


def _tc_layer(spmv, h2n, c2, es, wt, b, gamma, beta, wt_o=None, b_o=None):
    blk = lambda r, c: pl.BlockSpec((r, c), lambda i: (i, 0))
    blk_hi = lambda r, c: pl.BlockSpec((r, c), lambda i: (i + _GRID, 0))
    full = lambda r, c: pl.BlockSpec((r, c), lambda i: (0, 0))
    last = wt_o is not None
    in_specs = [
        blk(_R, _HALF), blk_hi(_R, _HALF),
        blk(_R, _HALF), blk_hi(_R, _HALF),
        blk(_R, _HALF), blk_hi(_R, _HALF),
        full(_NCODE, _ET), full(_ET, _D), full(_D, _D), full(1, _D),
        full(1, _D), full(1, _D),
    ]
    args = [spmv, spmv, h2n, h2n, c2, c2, jnp.asarray(_SEL), es, wt, b,
            gamma, beta]
    if last:
        in_specs += [full(_D, _D), full(1, _D)]
        args += [wt_o, b_o]
        out_spec = full(_N, _D)
        out_shape = jax.ShapeDtypeStruct((_N, _D), _f32)
        body = _tc_final_body
    else:
        out_spec = full(_NCORE * _N, _HALF)
        out_shape = jax.ShapeDtypeStruct((_NCORE * _N, _HALF), _f32)
        body = _tc_layer_body
    return pl.pallas_call(
        body,
        grid=(_GRID,),
        in_specs=in_specs,
        out_specs=out_spec,
        out_shape=out_shape,
        scratch_shapes=[
            pltpu.VMEM((_N, _D), _f32),
            pltpu.VMEM((2, _D), _f32),
        ],
    )(*args)


def kernel(x, edge_index, edge_attr, params):
    src = edge_index[0]
    dst = edge_index[1]
    ea = edge_attr.astype(jnp.int32)
    code = (ea[:, 0] + 2 * ea[:, 1] + 4 * ea[:, 2]
            + 8 * ea[:, 3] + 16 * ea[:, 4])

    zer_half = jnp.zeros((_RS0, _HALF), _f32)

    sc_spmv, sc_counts = _get_sc_kernels()
    id_rep = jnp.tile(jnp.eye(_HALF, dtype=_f32)[:_NCODE], (32, 1))
    c2 = sc_counts(code, dst, zer_half, id_rep)
    zer_dep = zer_half + c2[0, 0] * 0.0

    h2n = jnp.concatenate([x[:, :_HALF], x[:, _HALF:]], axis=0)
    out = None
    for li, lp in enumerate(params['layers']):
        es = jnp.concatenate(lp['embs'] + [jnp.zeros((1, _D), _f32)], axis=0)
        wt = lp['W'].T
        b = lp['b'].reshape(1, _D)
        gamma = lp['gamma'].reshape(1, _D)
        beta = lp['beta'].reshape(1, _D)
        spmv = sc_spmv(src, dst, h2n, zer_dep if li == 0 else zer_half)
        if li == len(params['layers']) - 1:
            out = _tc_layer(spmv, h2n, c2, es, wt, b, gamma, beta,
                            params['W_out'].T,
                            params['b_out'].reshape(1, _D))
        else:
            h2n = _tc_layer(spmv, h2n, c2, es, wt, b, gamma, beta)
    return out


# speedup vs baseline: 1.0164x; 1.0164x over previous
import functools

import numpy as np
import jax
import jax.numpy as jnp
from jax import lax
from jax.experimental import pallas as pl
from jax.experimental.pallas import tpu as pltpu
from jax.experimental.pallas import tpu_sc as plsc

_N = 10000
_E = 160000
_D = 256
_HALF = 128
_NCODE = 32
_CHUNK = 128
_NCHUNK = _E // _CHUNK
_NSUB = 16
_NCORE = 2
_RS0 = 632
_RSLAST = _N - (_NSUB - 1) * _RS0
_R = 1000
_GRID = _N // _R
_BOND_ROWS = [7, 7, 3, 3, 3]
_ET = 24

_EPS = 1e-5


def _make_selector() -> np.ndarray:
    off = np.cumsum([0] + _BOND_ROWS[:-1])
    s = np.zeros((_NCODE, _ET), np.float32)
    for c in range(_NCODE):
        for i in range(5):
            s[c, off[i] + ((c >> i) & 1)] += 1.0
    return s


_SEL = _make_selector()

_f32 = jnp.float32


def _zero_accum(sid, zer_hbm, accum, r0=_RS0, rlast=_RSLAST):
    start = pl.multiple_of(sid * r0, 8)

    @pl.when(sid < _NSUB - 1)
    def _():
        pltpu.sync_copy(zer_hbm.at[pl.ds(0, r0)], accum.at[pl.ds(start, r0)])

    @pl.when(sid == _NSUB - 1)
    def _():
        pltpu.sync_copy(zer_hbm.at[pl.ds(0, rlast)],
                        accum.at[pl.ds(start, rlast)])


def _copy_out(sid, base, accum, out_hbm, r0=_RS0, rlast=_RSLAST):
    start = pl.multiple_of(sid * r0, 8)
    dst0 = pl.multiple_of(base + sid * r0, 8)

    @pl.when(sid < _NSUB - 1)
    def _():
        pltpu.sync_copy(accum.at[pl.ds(start, r0)],
                        out_hbm.at[pl.ds(dst0, r0)])

    @pl.when(sid == _NSUB - 1)
    def _():
        pltpu.sync_copy(accum.at[pl.ds(start, rlast)],
                        out_hbm.at[pl.ds(dst0, rlast)])


_NB_S = 3
_CHUNK_S = 128
_NB_C = 3


def _ring_loop(nb, stride, wid, a_hbm, b_hbm, table_hbm, acc,
               abuf, bbuf, sbuf, rbuf, asem, bsem, gsem, ssem,
               transform_a, transform_b, chunk=_CHUNK):
    nchunk = _E // chunk
    nouter = nchunk // (nb * stride)

    def outer(k, carry):
        def cbase(b):
            return ((k * nb + b) * stride + wid) * chunk

        for b in range(nb):
            pltpu.async_copy(a_hbm.at[pl.ds(cbase(b), chunk)],
                             abuf[b], asem[b])
            pltpu.async_copy(b_hbm.at[pl.ds(cbase(b), chunk)],
                             bbuf[b], bsem[b])
        for b in range(nb):
            pltpu.make_async_copy(a_hbm.at[pl.ds(cbase(b), chunk)],
                                  abuf[b], asem[b]).wait()
            pltpu.make_async_copy(b_hbm.at[pl.ds(cbase(b), chunk)],
                                  bbuf[b], bsem[b]).wait()
            transform_a(abuf[b], bbuf[b])
            @pl.when(k > 0)
            def _(b=b):
                pltpu.make_async_copy(rbuf[b], acc.at[sbuf[b]],
                                      ssem[b]).wait()
            pltpu.async_copy(table_hbm.at[abuf[b]], rbuf[b], gsem[b])
        for b in range(nb):
            pltpu.make_async_copy(table_hbm.at[abuf[b]], rbuf[b],
                                  gsem[b]).wait()
            transform_b(bbuf[b])
            for j in range(chunk // 16):
                sl = pl.ds(j * 16, 16)
                sbuf[b][sl] = bbuf[b][sl]
            pltpu.async_copy(rbuf[b], acc.at[sbuf[b]], ssem[b], add=True)
        return carry

    lax.fori_loop(0, nouter, outer, 0)
    for b in range(nb):
        pltpu.make_async_copy(rbuf[b], acc.at[sbuf[b]], ssem[b]).wait()

    tail = nchunk - nouter * nb * stride

    @pl.when(wid < tail)
    def _():
        base = (nouter * nb * stride + wid) * chunk
        pltpu.sync_copy(a_hbm.at[pl.ds(base, chunk)], abuf[0])
        pltpu.sync_copy(b_hbm.at[pl.ds(base, chunk)], bbuf[0])
        transform_a(abuf[0], bbuf[0])
        pltpu.async_copy(table_hbm.at[abuf[0]], rbuf[0], gsem[0]).wait()
        transform_b(bbuf[0])
        pltpu.sync_copy(rbuf[0], acc.at[bbuf[0]], add=True)


def _sc_spmv_body(src_hbm, dst_hbm, h2n_hbm, zer_hbm, out_hbm, *scr):
    nb = _NB_S
    abuf, bbuf, sbuf = scr[:nb], scr[nb:2 * nb], scr[2 * nb:3 * nb]
    rbuf = scr[3 * nb:4 * nb]
    accum = scr[4 * nb]
    sems = scr[4 * nb + 1:]
    asem, bsem = sems[:nb], sems[nb:2 * nb]
    gsem, ssem = sems[2 * nb:3 * nb], sems[3 * nb:4 * nb]

    cid = lax.axis_index("c")
    sid = lax.axis_index("s")
    _zero_accum(sid, zer_hbm, accum)
    plsc.subcore_barrier()

    row_off = cid * _N

    def add_off(a, b_unused):
        for j in range(_CHUNK_S // 16):
            sl = pl.ds(j * 16, 16)
            a[sl] = a[sl] + row_off

    _ring_loop(nb, _NSUB, sid, src_hbm, dst_hbm, h2n_hbm, accum,
               abuf, bbuf, sbuf, rbuf, asem, bsem, gsem, ssem,
               add_off, lambda b: None, chunk=_CHUNK_S)

    plsc.subcore_barrier()
    _copy_out(sid, cid * _N, accum, out_hbm)


_sc_cache = {}


def _get_sc_kernels():
    if 'spmv' not in _sc_cache:
        mesh = plsc.VectorSubcoreMesh(
            core_axis_name="c", subcore_axis_name="s",
            num_cores=_NCORE, num_subcores=_NSUB)
        _sc_cache['spmv'] = functools.partial(
            pl.kernel,
            out_type=jax.ShapeDtypeStruct((_NCORE * _N, _HALF), _f32),
            mesh=mesh,
            scratch_types=(
                [pltpu.VMEM((_CHUNK_S,), jnp.int32)] * (3 * _NB_S)
                + [pltpu.VMEM((_CHUNK_S, _HALF), _f32)] * _NB_S
                + [pltpu.VMEM_SHARED((_N, _HALF), _f32)]
                + [pltpu.SemaphoreType.DMA] * (4 * _NB_S)
            ),
        )(_sc_spmv_body)
        _sc_cache['counts'] = functools.partial(
            pl.kernel,
            out_type=jax.ShapeDtypeStruct((_NCORE * _N, _HALF), _f32),
            mesh=mesh,
            scratch_types=(
                [pltpu.VMEM((_CHUNK,), jnp.int32)] * (3 * _NB_C)
                + [pltpu.VMEM((_CHUNK, _HALF), _f32)] * _NB_C
                + [pltpu.VMEM_SHARED((_N, _HALF), _f32)]
                + [pltpu.SemaphoreType.DMA] * (4 * _NB_C)
            ),
        )(_sc_counts_body)
    return _sc_cache['spmv'], _sc_cache['counts']


_QROWS = 2504
_QR0 = 160
_QRLAST = _QROWS - (_NSUB - 1) * _QR0


def _sc_counts_body(code_hbm, dst_hbm, zer_hbm, id128_hbm, out_hbm, *scr):
    nb = _NB_C
    abuf, bbuf, sbuf = scr[:nb], scr[nb:2 * nb], scr[2 * nb:3 * nb]
    rbuf = scr[3 * nb:4 * nb]
    accum = scr[4 * nb]
    sems = scr[4 * nb + 1:]
    asem, bsem = sems[:nb], sems[nb:2 * nb]
    gsem, ssem = sems[2 * nb:3 * nb], sems[3 * nb:4 * nb]

    cid = lax.axis_index("c")
    sid = lax.axis_index("s")
    _zero_accum(sid, zer_hbm, accum)
    plsc.subcore_barrier()

    wid = sid * _NCORE + cid
    tab_off = wid * _NCODE

    def tf_a(a, b):
        for j in range(_CHUNK // 16):
            sl = pl.ds(j * 16, 16)
            a[sl] = a[sl] + tab_off

    _ring_loop(nb, _NSUB * _NCORE, wid, code_hbm, dst_hbm, id128_hbm, accum,
               abuf, bbuf, sbuf, rbuf, asem, bsem, gsem, ssem,
               tf_a, lambda b: None)

    plsc.subcore_barrier()
    _copy_out(sid, cid * _N, accum, out_hbm)


def _bn_relu(y, stats, gamma, beta):
    mu = stats[0:1, :] * (1.0 / _N)
    var = stats[1:2, :] * (1.0 / _N) - mu * mu
    return jnp.maximum(gamma * (y - mu) * lax.rsqrt(var + _EPS) + beta, 0.0)


def _layer_common(i, sp_lo, sp_hi, h_lo, h_hi, c_lo, c_hi, sel, es, wt, b,
                  y_scr, st_scr):
    t = jnp.dot(sel[...], es[...], preferred_element_type=_f32,
                precision=lax.Precision.HIGHEST)
    cb = c_lo[...][:, :_NCODE] + c_hi[...][:, :_NCODE]
    emb = jnp.dot(cb, t, preferred_element_type=_f32,
                  precision=lax.Precision.HIGHEST) + t[0:1, :]
    aggr = jnp.concatenate(
        [sp_lo[...] + h_lo[...], sp_hi[...] + h_hi[...]], axis=1) + emb
    y = jnp.dot(aggr.astype(jnp.bfloat16), wt[...].astype(jnp.bfloat16),
                preferred_element_type=_f32) + b[...]
    y_scr[pl.ds(pl.multiple_of(i * _R, 8), _R), :] = y
    st = jnp.concatenate(
        [jnp.sum(y, axis=0, keepdims=True),
         jnp.sum(y * y, axis=0, keepdims=True)], axis=0)

    @pl.when(i == 0)
    def _():
        st_scr[...] = st

    @pl.when(i > 0)
    def _():
        st_scr[...] += st


def _tc_layer_body(sp_lo, sp_hi, h_lo, h_hi, c_lo, c_hi, sel, es, wt, b,
                   gamma, beta, out, y_scr, st_scr):
    i = pl.program_id(0)
    _layer_common(i, sp_lo, sp_hi, h_lo, h_hi, c_lo, c_hi, sel, es, wt, b,
                  y_scr, st_scr)

    @pl.when(i == _GRID - 1)
    def _():
        def norm_blk(j, carry):
            r0 = pl.multiple_of(j * _R, 8)
            h = _bn_relu(y_scr[pl.ds(r0, _R), :], st_scr[...],
                         gamma[...], beta[...])
            out[pl.ds(r0, _R), :] = h[:, :_HALF]
            out[pl.ds(_N + r0, _R), :] = h[:, _HALF:]
            return carry

        lax.fori_loop(0, _GRID, norm_blk, 0)


def _tc_final_body(sp_lo, sp_hi, h_lo, h_hi, c_lo, c_hi, sel, es, wt, b,
                   gamma, beta, wt_o, b_o, out, y_scr, st_scr):
    i = pl.program_id(0)
    _layer_common(i, sp_lo, sp_hi, h_lo, h_hi, c_lo, c_hi, sel, es, wt, b,
                  y_scr, st_scr)

    @pl.when(i == _GRID - 1)
    def _():
        def norm_blk(j, carry):
            r0 = pl.multiple_of(j * _R, 8)
            h = _bn_relu(y_scr[pl.ds(r0, _R), :], st_scr[...],
                         gamma[...], beta[...])
            out[pl.ds(r0, _R), :] = jnp.dot(
                h.astype(jnp.bfloat16), wt_o[...].astype(jnp.bfloat16),
                preferred_element_type=_f32) + b_o[...]
            return carry

        lax.fori_loop(0, _GRID, norm_blk, 0)


def _tc_layer(spmv, h2n, c2, es, wt, b, gamma, beta, wt_o=None, b_o=None):
    blk = lambda r, c: pl.BlockSpec((r, c), lambda i: (i, 0))
    blk_hi = lambda r, c: pl.BlockSpec((r, c), lambda i: (i + _GRID, 0))
    full = lambda r, c: pl.BlockSpec((r, c), lambda i: (0, 0))
    last = wt_o is not None
    in_specs = [
        blk(_R, _HALF), blk_hi(_R, _HALF),
        blk(_R, _HALF), blk_hi(_R, _HALF),
        blk(_R, _HALF), blk_hi(_R, _HALF),
        full(_NCODE, _ET), full(_ET, _D), full(_D, _D), full(1, _D),
        full(1, _D), full(1, _D),
    ]
    args = [spmv, spmv, h2n, h2n, c2, c2, jnp.asarray(_SEL), es, wt, b,
            gamma, beta]
    if last:
        in_specs += [full(_D, _D), full(1, _D)]
        args += [wt_o, b_o]
        out_spec = full(_N, _D)
        out_shape = jax.ShapeDtypeStruct((_N, _D), _f32)
        body = _tc_final_body
    else:
        out_spec = full(_NCORE * _N, _HALF)
        out_shape = jax.ShapeDtypeStruct((_NCORE * _N, _HALF), _f32)
        body = _tc_layer_body
    return pl.pallas_call(
        body,
        grid=(_GRID,),
        in_specs=in_specs,
        out_specs=out_spec,
        out_shape=out_shape,
        scratch_shapes=[
            pltpu.VMEM((_N, _D), _f32),
            pltpu.VMEM((2, _D), _f32),
        ],
    )(*args)


def kernel(x, edge_index, edge_attr, params):
    src = edge_index[0]
    dst = edge_index[1]
    ea = edge_attr.astype(jnp.int32)
    code = (ea[:, 0] + 2 * ea[:, 1] + 4 * ea[:, 2]
            + 8 * ea[:, 3] + 16 * ea[:, 4])

    zer_half = jnp.zeros((_RS0, _HALF), _f32)

    sc_spmv, sc_counts = _get_sc_kernels()
    id_rep = jnp.tile(jnp.eye(_HALF, dtype=_f32)[:_NCODE], (32, 1))
    c2 = sc_counts(code, dst, zer_half, id_rep)
    zer_dep = zer_half + c2[0, 0] * 0.0

    h2n = jnp.concatenate([x[:, :_HALF], x[:, _HALF:]], axis=0)
    out = None
    for li, lp in enumerate(params['layers']):
        es = jnp.concatenate(lp['embs'] + [jnp.zeros((1, _D), _f32)], axis=0)
        wt = lp['W'].T
        b = lp['b'].reshape(1, _D)
        gamma = lp['gamma'].reshape(1, _D)
        beta = lp['beta'].reshape(1, _D)
        spmv = sc_spmv(src, dst, h2n, zer_dep if li == 0 else zer_half)
        if li == len(params['layers']) - 1:
            out = _tc_layer(spmv, h2n, c2, es, wt, b, gamma, beta,
                            params['W_out'].T,
                            params['b_out'].reshape(1, _D))
        else:
            h2n = _tc_layer(spmv, h2n, c2, es, wt, b, gamma, beta)
    return out
